# baseline (speedup 1.0000x reference)
import functools

import jax
import jax.numpy as jnp
from jax import lax
from jax.experimental import pallas as pl
from jax.experimental.pallas import tpu as pltpu
from jax.experimental.pallas import tpu_sc as plsc

_info = plsc.get_sparse_core_info()
_NC = _info.num_cores
_NS = _info.num_subcores
_NW = _NC * _NS

_BLK = 128
_GRP = 8
_NWIDE = 4
_PADW = 128


def _make_body(seq, emb):

    def body(gidx_hbm, cidx_hbm, gtab_hbm, ctab_hbm, out_hbm,
             gidx_v, cidx_v, wide_v, tr_v, ctab_sh,
             gsem0, gsem1, asem0, asem1, osem0, osem1, isem):
        gsems = (gsem0, gsem1)
        asems = (asem0, asem1)
        osems = (osem0, osem1)
        sid = lax.axis_index("s")
        wid = sid * _NC + lax.axis_index("c")
        col0 = wid * _BLK

        @pl.when(sid == 0)
        def _stage():
            pltpu.sync_copy(ctab_hbm, ctab_sh)
        plsc.subcore_barrier()

        def idx_start(g, ib):
            pltpu.async_copy(
                gidx_hbm.at[pl.ds(g * _GRP, _GRP), pl.ds(col0, _BLK)],
                gidx_v.at[ib], isem)
            pltpu.async_copy(
                cidx_hbm.at[pl.ds(g * _GRP, _GRP), pl.ds(col0, _BLK)],
                cidx_v.at[ib], isem)

        def idx_wait(ib):
            pltpu.make_async_copy(
                gidx_hbm.at[pl.ds(0, _GRP), pl.ds(col0, _BLK)],
                gidx_v.at[ib], isem).wait()
            pltpu.make_async_copy(
                cidx_hbm.at[pl.ds(0, _GRP), pl.ds(col0, _BLK)],
                cidx_v.at[ib], isem).wait()

        def gene_start(ib, q, w, gsem):
            pltpu.async_copy(gtab_hbm.at[gidx_v.at[ib, q]],
                             wide_v.at[w], gsem)

        def gene_wait(w, gsem):
            pltpu.make_async_copy(gtab_hbm.at[gidx_v.at[0, 0]],
                                  wide_v.at[w], gsem).wait()

        def add_start(ib, q, w, asem):
            pltpu.async_copy(ctab_sh.at[cidx_v.at[ib, q]],
                             wide_v.at[w], asem, add=True)

        def add_wait(w, asem):
            pltpu.make_async_copy(ctab_sh.at[cidx_v.at[0, 0]],
                                  wide_v.at[w], asem).wait()

        rows16 = [jnp.arange(16, dtype=jnp.int32) + b0
                  for b0 in range(0, _BLK, 16)]

        def transpose(w, c):
            def col(e, carry):
                cols = jnp.full((16,), e, dtype=jnp.int32)
                for j in range(_BLK // 16):
                    vec = plsc.load_gather(wide_v.at[w], [rows16[j], cols])
                    tr_v[c, e, pl.ds(j * 16, 16)] = vec
                return carry
            lax.fori_loop(0, emb, col, 0)

        def write_start(s, c, osem):
            pltpu.async_copy(
                tr_v.at[c],
                out_hbm.at[s, slice(None), pl.ds(col0, _BLK)],
                osem)

        def write_wait(c, osem):
            pltpu.make_async_copy(
                tr_v.at[c],
                out_hbm.at[0, slice(None), pl.ds(col0, _BLK)],
                osem).wait()

        idx_start(0, 0)
        idx_wait(0)
        gene_start(0, 0, 0, gsems[0])

        ngroups = seq // _GRP

        def group(g, carry):
            ib = lax.rem(g, 2)
            ibn = lax.rem(g + 1, 2)
            for q in range(_GRP):
                s = g * _GRP + q
                w = q % _NWIDE
                p = q % 2
                c = q % 2

                if q == 1:
                    @pl.when(g + 1 < ngroups)
                    def _pref_idx():
                        idx_start(g + 1, ibn)

                @pl.when(s + 1 < seq)
                def _pref(s=s, q=q, w=w, p=p):
                    qn = (q + 1) % _GRP
                    if qn == 0:
                        idx_wait(ibn)
                    gene_start(ibn if qn == 0 else ib, qn,
                               (w + 1) % _NWIDE, gsems[1 - p])

                gene_wait(w, gsems[p])
                add_start(ib, q, w, asems[p])

                @pl.when(s >= 1)
                def _retire(s=s, w=w, p=p, c=c):
                    wp = (w - 1) % _NWIDE
                    add_wait(wp, asems[1 - p])
                    @pl.when(s >= 3)
                    def _free():
                        write_wait(1 - c, osems[1 - c])
                    transpose(wp, 1 - c)
                    write_start(s - 1, 1 - c, osems[1 - c])
            return carry

        lax.fori_loop(0, ngroups, group, 0)

        last = seq - 1
        wl = last % _NWIDE
        cl = last % 2
        add_wait(wl, asems[last % 2])
        write_wait(cl, osems[cl])
        transpose(wl, cl)
        write_start(last, cl, osems[cl])
        for c in range(2):
            write_wait(c, osems[c])

    return body


@functools.partial(jax.jit, static_argnums=())
def _embedding_sum(gidxT, cidxT, gtab, ctab):
    seq, batch = gidxT.shape
    emb = 64
    cvocab = ctab.shape[0]

    body = _make_body(seq, emb)
    call = pl.kernel(
        body,
        out_type=jax.ShapeDtypeStruct((seq, emb, batch), jnp.float32),
        scratch_types=[
            pltpu.VMEM((2, _GRP, _BLK), jnp.int32),
            pltpu.VMEM((2, _GRP, _BLK), jnp.int32),
            pltpu.VMEM((_NWIDE, _BLK, _PADW), jnp.float32),
            pltpu.VMEM((2, emb, _BLK), jnp.float32),
            pltpu.VMEM_SHARED((cvocab, _PADW), jnp.float32),
        ] + [pltpu.SemaphoreType.DMA] * 7,
        mesh=plsc.VectorSubcoreMesh(core_axis_name="c", subcore_axis_name="s"),
        compiler_params=pltpu.CompilerParams(use_tc_tiling_on_sc=True,
                                             needs_layout_passes=False),
    )
    return call(gidxT, cidxT, gtab, ctab)


def kernel(gene_id, count_id, gene_table, count_table):
    emb = gene_table.shape[1]
    gidxT = gene_id.T.astype(jnp.int32)
    cidxT = count_id.T.astype(jnp.int32)
    gtab = jnp.pad(gene_table, ((0, 0), (0, _PADW - emb)))
    ctab = jnp.pad(count_table, ((0, 0), (0, _PADW - emb)))
    out = _embedding_sum(gidxT, cidxT, gtab, ctab)
    return jnp.transpose(out, (2, 0, 1))


# speedup vs baseline: 2.6566x; 2.6566x over previous
import functools

import jax
import jax.numpy as jnp
from jax import lax
from jax.experimental import pallas as pl
from jax.experimental.pallas import tpu as pltpu
from jax.experimental.pallas import tpu_sc as plsc

_info = plsc.get_sparse_core_info()
_NC = _info.num_cores
_NS = _info.num_subcores
_NW = _NC * _NS

_BLK = 128
_GRP = 8
_NWIDE = 4
_PADW = 128


def _make_body(seq, emb):

    def body(gidx_hbm, cidx_hbm, gtab_hbm, ctab_hbm, out_hbm,
             gidx_v, cidx_v, wide_v, tr_v, ctab_sh,
             gsem0, gsem1, asem0, asem1, osem0, osem1, isem):
        gsems = (gsem0, gsem1)
        asems = (asem0, asem1)
        osems = (osem0, osem1)
        sid = lax.axis_index("s")
        wid = sid * _NC + lax.axis_index("c")
        col0 = wid * _BLK

        @pl.when(sid == 0)
        def _stage():
            pltpu.sync_copy(ctab_hbm, ctab_sh)
        plsc.subcore_barrier()

        def idx_start(g, ib):
            pltpu.async_copy(
                gidx_hbm.at[pl.ds(g * _GRP, _GRP), pl.ds(col0, _BLK)],
                gidx_v.at[ib], isem)
            pltpu.async_copy(
                cidx_hbm.at[pl.ds(g * _GRP, _GRP), pl.ds(col0, _BLK)],
                cidx_v.at[ib], isem)

        def idx_wait(ib):
            pltpu.make_async_copy(
                gidx_hbm.at[pl.ds(0, _GRP), pl.ds(col0, _BLK)],
                gidx_v.at[ib], isem).wait()
            pltpu.make_async_copy(
                cidx_hbm.at[pl.ds(0, _GRP), pl.ds(col0, _BLK)],
                cidx_v.at[ib], isem).wait()

        def gene_start(ib, q, w, gsem):
            pltpu.async_copy(gtab_hbm.at[gidx_v.at[ib, q]],
                             wide_v.at[w], gsem)

        def gene_wait(w, gsem):
            pltpu.make_async_copy(gtab_hbm.at[gidx_v.at[0, 0]],
                                  wide_v.at[w], gsem).wait()

        def add_start(ib, q, w, asem):
            pltpu.async_copy(ctab_sh.at[cidx_v.at[ib, q]],
                             wide_v.at[w], asem, add=True)

        def add_wait(w, asem):
            pltpu.make_async_copy(ctab_sh.at[cidx_v.at[0, 0]],
                                  wide_v.at[w], asem).wait()

        iota16 = jnp.arange(16, dtype=jnp.int32)
        diag16 = [(iota16 + k) % 16 for k in range(16)]

        def transpose(w, c):
            def block(blk, carry):
                b0 = lax.rem(blk, _BLK // 16) * 16
                e0 = lax.div(blk, _BLK // 16) * 16
                rows = b0 + iota16
                for k in range(16):
                    cols = e0 + diag16[k]
                    vec = plsc.load_gather(wide_v.at[w], [rows, cols])
                    plsc.store_scatter(tr_v.at[c], [cols, rows], vec)
                return carry
            lax.fori_loop(0, (_BLK // 16) * (emb // 16), block, 0)

        def write_start(s, c, osem):
            pltpu.async_copy(
                tr_v.at[c],
                out_hbm.at[s, slice(None), pl.ds(col0, _BLK)],
                osem)

        def write_wait(c, osem):
            pltpu.make_async_copy(
                tr_v.at[c],
                out_hbm.at[0, slice(None), pl.ds(col0, _BLK)],
                osem).wait()

        idx_start(0, 0)
        idx_wait(0)
        gene_start(0, 0, 0, gsems[0])

        ngroups = seq // _GRP

        def group(g, carry):
            ib = lax.rem(g, 2)
            ibn = lax.rem(g + 1, 2)
            for q in range(_GRP):
                s = g * _GRP + q
                w = q % _NWIDE
                p = q % 2
                c = q % 2

                if q == 1:
                    @pl.when(g + 1 < ngroups)
                    def _pref_idx():
                        idx_start(g + 1, ibn)

                @pl.when(s + 1 < seq)
                def _pref(s=s, q=q, w=w, p=p):
                    qn = (q + 1) % _GRP
                    if qn == 0:
                        idx_wait(ibn)
                    gene_start(ibn if qn == 0 else ib, qn,
                               (w + 1) % _NWIDE, gsems[1 - p])

                gene_wait(w, gsems[p])
                add_start(ib, q, w, asems[p])

                @pl.when(s >= 1)
                def _retire(s=s, w=w, p=p, c=c):
                    wp = (w - 1) % _NWIDE
                    add_wait(wp, asems[1 - p])
                    @pl.when(s >= 3)
                    def _free():
                        write_wait(1 - c, osems[1 - c])
                    transpose(wp, 1 - c)
                    write_start(s - 1, 1 - c, osems[1 - c])
            return carry

        lax.fori_loop(0, ngroups, group, 0)

        last = seq - 1
        wl = last % _NWIDE
        cl = last % 2
        add_wait(wl, asems[last % 2])
        write_wait(cl, osems[cl])
        transpose(wl, cl)
        write_start(last, cl, osems[cl])
        for c in range(2):
            write_wait(c, osems[c])

    return body


@functools.partial(jax.jit, static_argnums=())
def _embedding_sum(gidxT, cidxT, gtab, ctab):
    seq, batch = gidxT.shape
    emb = 64
    cvocab = ctab.shape[0]

    body = _make_body(seq, emb)
    call = pl.kernel(
        body,
        out_type=jax.ShapeDtypeStruct((seq, emb, batch), jnp.float32),
        scratch_types=[
            pltpu.VMEM((2, _GRP, _BLK), jnp.int32),
            pltpu.VMEM((2, _GRP, _BLK), jnp.int32),
            pltpu.VMEM((_NWIDE, _BLK, _PADW), jnp.float32),
            pltpu.VMEM((2, emb, _BLK), jnp.float32),
            pltpu.VMEM_SHARED((cvocab, _PADW), jnp.float32),
        ] + [pltpu.SemaphoreType.DMA] * 7,
        mesh=plsc.VectorSubcoreMesh(core_axis_name="c", subcore_axis_name="s"),
        compiler_params=pltpu.CompilerParams(use_tc_tiling_on_sc=True,
                                             needs_layout_passes=False),
    )
    return call(gidxT, cidxT, gtab, ctab)


def kernel(gene_id, count_id, gene_table, count_table):
    emb = gene_table.shape[1]
    gidxT = gene_id.T.astype(jnp.int32)
    cidxT = count_id.T.astype(jnp.int32)
    gtab = jnp.pad(gene_table, ((0, 0), (0, _PADW - emb)))
    ctab = jnp.pad(count_table, ((0, 0), (0, _PADW - emb)))
    out = _embedding_sum(gidxT, cidxT, gtab, ctab)
    return jnp.transpose(out, (2, 0, 1))
